# baseline (speedup 1.0000x reference)
import jax
import jax.numpy as jnp
from jax.experimental import pallas as pl
from jax.experimental.pallas import tpu as pltpu

_HI = jax.lax.Precision.HIGHEST


def _gcn_kernel(adj_ref, x0_ref, fcw_ref, fcb_ref, w0_ref, b0_ref, w1_ref,
                b1_ref, wp_ref, bp_ref, out_ref, s0_ref, s1_ref):
    p = pl.program_id(0)
    i = pl.program_id(1)
    bm = adj_ref.shape[0]

    @pl.when(jnp.logical_and(p == 0, i == 0))
    def _prologue():
        x = jnp.dot(x0_ref[...], fcw_ref[...],
                    preferred_element_type=jnp.float32,
                    precision=_HI) + fcb_ref[...]
        s0 = jnp.dot(x, w0_ref[...], preferred_element_type=jnp.float32,
                     precision=_HI)
        s0_ref[...] = s0.astype(jnp.bfloat16)

    a = adj_ref[...].astype(jnp.bfloat16)

    @pl.when(p == 0)
    def _pass_a():
        t = jnp.dot(a, s0_ref[...], preferred_element_type=jnp.float32)
        h = jnp.maximum(t + b0_ref[...], 0.0)
        s1 = jnp.dot(h, w1_ref[...], preferred_element_type=jnp.float32,
                     precision=_HI)
        s1_ref[pl.ds(i * bm, bm), :] = s1.astype(jnp.bfloat16)
        out_ref[...] = jnp.zeros_like(out_ref)

    @pl.when(p == 1)
    def _pass_b():
        t = jnp.dot(a, s1_ref[...], preferred_element_type=jnp.float32)
        h = jnp.maximum(t + b1_ref[...], 0.0)
        out_ref[...] = jnp.dot(h, wp_ref[...],
                               preferred_element_type=jnp.float32,
                               precision=_HI) + bp_ref[...]


def _pick_block(n):
    for bm in (200, 400, 500, 250, 125, 100, 80, 50, 40, 25, 20, 16, 10, 8, 5, 4, 2, 1):
        if n % bm == 0:
            return bm
    return n


def kernel(X, adj, fc_W, fc_b, conv0_W, conv0_b, conv1_W, conv1_b, pred_W, pred_b):
    x0 = X[0]
    n, f_in = x0.shape
    h_dim = conv0_W.shape[1]
    out_dim = conv1_W.shape[1]
    c_dim = pred_W.shape[1]
    bm = _pick_block(n)
    grid = (2, n // bm)

    whole = lambda shape: pl.BlockSpec(shape, lambda p, i: (0, 0))

    out = pl.pallas_call(
        _gcn_kernel,
        grid=grid,
        in_specs=[
            pl.BlockSpec((bm, n), lambda p, i: (i, 0)),
            whole((n, f_in)),
            whole((f_in, f_in)),
            whole((1, f_in)),
            whole((f_in, h_dim)),
            whole((1, h_dim)),
            whole((h_dim, out_dim)),
            whole((1, out_dim)),
            whole((out_dim, c_dim)),
            whole((1, c_dim)),
        ],
        out_specs=pl.BlockSpec((bm, c_dim), lambda p, i: (i, 0)),
        out_shape=jax.ShapeDtypeStruct((n, c_dim), jnp.float32),
        scratch_shapes=[
            pltpu.VMEM((n, h_dim), jnp.bfloat16),
            pltpu.VMEM((n, out_dim), jnp.bfloat16),
        ],
    )(adj, x0, fc_W, fc_b.reshape(1, -1), conv0_W, conv0_b.reshape(1, -1),
      conv1_W, conv1_b.reshape(1, -1), pred_W, pred_b.reshape(1, -1))

    return out


# speedup vs baseline: 1.1646x; 1.1646x over previous
import jax
import jax.numpy as jnp
from jax.experimental import pallas as pl
from jax.experimental.pallas import tpu as pltpu

_HI = jax.lax.Precision.HIGHEST


def _proj_kernel(x_ref, fcw_ref, fcb_ref, w0_ref, s0_ref):
    wa = jnp.dot(fcw_ref[...], w0_ref[...], preferred_element_type=jnp.float32,
                 precision=_HI)
    c = jnp.dot(fcb_ref[...], w0_ref[...], preferred_element_type=jnp.float32,
                precision=_HI)
    x = x_ref[...]
    xh = x.astype(jnp.bfloat16)
    xl = (x - xh.astype(jnp.float32)).astype(jnp.bfloat16)
    wh = wa.astype(jnp.bfloat16)
    wl = (wa - wh.astype(jnp.float32)).astype(jnp.bfloat16)
    s0 = (jnp.dot(xh, wh, preferred_element_type=jnp.float32)
          + jnp.dot(xh, wl, preferred_element_type=jnp.float32)
          + jnp.dot(xl, wh, preferred_element_type=jnp.float32)) + c
    s0_ref[...] = s0.astype(jnp.bfloat16)


def _main_kernel(adj_ref, s0_ref, b0_ref, w1_ref, b1_ref, wp_ref, bp_ref,
                 out_ref, s1_ref):
    p = pl.program_id(0)
    i = pl.program_id(1)
    bm = adj_ref.shape[0]
    a = adj_ref[...].astype(jnp.bfloat16)

    @pl.when(p == 0)
    def _pass_a():
        t = jnp.dot(a, s0_ref[...], preferred_element_type=jnp.float32)
        h = jnp.maximum(t + b0_ref[...], 0.0)
        s1 = jnp.dot(h, w1_ref[...], preferred_element_type=jnp.float32,
                     precision=_HI)
        s1_ref[pl.ds(i * bm, bm), :] = s1.astype(jnp.bfloat16)
        out_ref[...] = jnp.zeros_like(out_ref)

    @pl.when(p == 1)
    def _pass_b():
        t = jnp.dot(a, s1_ref[...], preferred_element_type=jnp.float32)
        h = jnp.maximum(t + b1_ref[...], 0.0)
        out_ref[...] = jnp.dot(h, wp_ref[...],
                               preferred_element_type=jnp.float32,
                               precision=_HI) + bp_ref[...]


def _pick_block(n):
    for bm in (400, 200, 100, 80, 40, 25, 20, 16, 10, 8, 5, 4, 2, 1):
        if n % bm == 0:
            return bm
    return n


def kernel(X, adj, fc_W, fc_b, conv0_W, conv0_b, conv1_W, conv1_b, pred_W, pred_b):
    x0 = X[0]
    n, f_in = x0.shape
    h_dim = conv0_W.shape[1]
    out_dim = conv1_W.shape[1]
    c_dim = pred_W.shape[1]
    bm = _pick_block(n)

    s0 = pl.pallas_call(
        _proj_kernel,
        out_shape=jax.ShapeDtypeStruct((n, h_dim), jnp.bfloat16),
    )(x0, fc_W, fc_b.reshape(1, -1), conv0_W)

    whole = lambda shape: pl.BlockSpec(shape, lambda p, i: (0, 0))

    out = pl.pallas_call(
        _main_kernel,
        grid=(2, n // bm),
        in_specs=[
            pl.BlockSpec((bm, n), lambda p, i: (i, 0)),
            whole((n, h_dim)),
            whole((1, h_dim)),
            whole((h_dim, out_dim)),
            whole((1, out_dim)),
            whole((out_dim, c_dim)),
            whole((1, c_dim)),
        ],
        out_specs=pl.BlockSpec((bm, c_dim), lambda p, i: (i, 0)),
        out_shape=jax.ShapeDtypeStruct((n, c_dim), jnp.float32),
        scratch_shapes=[
            pltpu.VMEM((n, out_dim), jnp.bfloat16),
        ],
    )(adj, s0, conv0_b.reshape(1, -1), conv1_W, conv1_b.reshape(1, -1),
      pred_W, pred_b.reshape(1, -1))

    return out


# speedup vs baseline: 1.2306x; 1.0566x over previous
import jax
import jax.numpy as jnp
from jax.experimental import pallas as pl
from jax.experimental.pallas import tpu as pltpu

_HI = jax.lax.Precision.HIGHEST

_NBUF = 4


def _make_kernel(n, f_in, h_dim, out_dim, c_dim, bm):
    nstripes = n // bm
    nsteps = 2 * nstripes
    ahead = _NBUF - 1

    def body(x_ref, adj_ref, fcw_ref, fcb_ref, w0_ref, b0_ref, w1_ref,
             b1_ref, wp_ref, bp_ref, out_ref, buf_ref, s0_ref, s1f_ref,
             s1_ref, sem):
        def issue(step, slot):
            stripe = jax.lax.rem(step, nstripes)
            pltpu.make_async_copy(
                adj_ref.at[pl.ds(stripe * bm, bm), :],
                buf_ref.at[slot],
                sem.at[slot],
            ).start()

        for j in range(ahead):
            issue(j, j)

        wa = jnp.dot(fcw_ref[...], w0_ref[...],
                     preferred_element_type=jnp.float32, precision=_HI)
        c = jnp.dot(fcb_ref[...], w0_ref[...],
                    preferred_element_type=jnp.float32, precision=_HI)
        wh = wa.astype(jnp.bfloat16)
        wl = (wa - wh.astype(jnp.float32)).astype(jnp.bfloat16)
        pchunk = 2000 if n % 2000 == 0 else bm
        for r in range(0, n, pchunk):
            x = x_ref[r:r + pchunk, :]
            xh = x.astype(jnp.bfloat16)
            xl = (x - xh.astype(jnp.float32)).astype(jnp.bfloat16)
            s0 = (jnp.dot(xh, wh, preferred_element_type=jnp.float32)
                  + jnp.dot(xh, wl, preferred_element_type=jnp.float32)
                  + jnp.dot(xl, wh, preferred_element_type=jnp.float32)) + c
            s0_ref[r:r + pchunk, :] = s0.astype(jnp.bfloat16)

        def step_fn(h, _):
            slot = jax.lax.rem(h, _NBUF)
            stripe = jax.lax.rem(h, nstripes)
            pltpu.make_async_copy(
                adj_ref.at[pl.ds(stripe * bm, bm), :],
                buf_ref.at[slot],
                sem.at[slot],
            ).wait()

            @pl.when(h + ahead < nsteps)
            def _issue_next():
                issue(h + ahead, jax.lax.rem(h + ahead, _NBUF))

            a = buf_ref[slot].astype(jnp.bfloat16)

            @pl.when(h < nstripes)
            def _pass_a():
                t = jnp.dot(a, s0_ref[...], preferred_element_type=jnp.float32)
                hh = jnp.maximum(t + b0_ref[...], 0.0)
                s1 = jnp.dot(hh, w1_ref[...],
                             preferred_element_type=jnp.float32, precision=_HI)
                s1f_ref[pl.ds(stripe * bm, bm), :] = s1

            @pl.when(h == nstripes)
            def _stage_s1():
                s1_ref[...] = s1f_ref[...].astype(jnp.bfloat16)

            @pl.when(h >= nstripes)
            def _pass_b():
                t = jnp.dot(a, s1_ref[...], preferred_element_type=jnp.float32)
                hh = jnp.maximum(t + b1_ref[...], 0.0)
                out_ref[pl.ds(stripe * bm, bm), :] = jnp.dot(
                    hh, wp_ref[...], preferred_element_type=jnp.float32,
                    precision=_HI) + bp_ref[...]

            return 0

        jax.lax.fori_loop(0, nsteps, step_fn, 0)

    return body


def _pick_block(n):
    for bm in (200, 400, 100, 80, 40, 25, 20, 16, 10, 8, 5, 4, 2, 1):
        if n % bm == 0 and n // bm >= _NBUF:
            return bm
    return n


def kernel(X, adj, fc_W, fc_b, conv0_W, conv0_b, conv1_W, conv1_b, pred_W, pred_b):
    x0 = X[0]
    n, f_in = x0.shape
    h_dim = conv0_W.shape[1]
    out_dim = conv1_W.shape[1]
    c_dim = pred_W.shape[1]
    bm = _pick_block(n)

    vmem = pl.BlockSpec(memory_space=pltpu.MemorySpace.VMEM)

    out = pl.pallas_call(
        _make_kernel(n, f_in, h_dim, out_dim, c_dim, bm),
        in_specs=[
            vmem,
            pl.BlockSpec(memory_space=pl.ANY),
            vmem, vmem, vmem, vmem, vmem, vmem, vmem, vmem,
        ],
        out_specs=vmem,
        out_shape=jax.ShapeDtypeStruct((n, c_dim), jnp.float32),
        scratch_shapes=[
            pltpu.VMEM((_NBUF, bm, n), jnp.float32),
            pltpu.VMEM((n, h_dim), jnp.bfloat16),
            pltpu.VMEM((n, out_dim), jnp.float32),
            pltpu.VMEM((n, out_dim), jnp.bfloat16),
            pltpu.SemaphoreType.DMA((_NBUF,)),
        ],
    )(x0, adj, fc_W, fc_b.reshape(1, -1), conv0_W, conv0_b.reshape(1, -1),
      conv1_W, conv1_b.reshape(1, -1), pred_W, pred_b.reshape(1, -1))

    return out
